# baseline (speedup 1.0000x reference)
import dataclasses
import functools

import jax
import jax.numpy as jnp
from jax import lax
from jax.experimental import pallas as pl
from jax.experimental.pallas import tpu as pltpu
from jax.experimental.pallas import tpu_sc as plsc

_NC = 2
_NS = 16
_LANES = 16
_STAGE = 8


def _sc_body(ts_hbm, nv_hbm, mk_hbm, wd_hbm, wv_hbm, bd_hbm, bv_hbm, tb_hbm,
             out_hbm, ts8, nv8, mk8, wdv, wvv, b1, b2, b3, biasv, plane,
             sems):
    B = ts_hbm.shape[0]
    D = wd_hbm.shape[0]
    L = ts_hbm.shape[1]
    nw = _NC * _NS
    per_w = B // nw
    wid = lax.axis_index("s") * _NC + lax.axis_index("c")
    base = wid * per_w

    pltpu.sync_copy(wd_hbm, wdv)
    pltpu.sync_copy(wv_hbm, wvv)
    pltpu.sync_copy(bd_hbm, b1)
    pltpu.sync_copy(bv_hbm, b2)
    pltpu.sync_copy(tb_hbm, b3)
    for c in range(D // _LANES):
        sl = pl.ds(c * _LANES, _LANES)
        biasv[sl] = b1[sl] + b2[sl] + b3[sl]

    offs = list(range(0, L - _LANES, _LANES)) + [L - _LANES]

    def row_body(bl, _):
        slot = lax.rem(bl, 2)

        @pl.when(lax.rem(bl, _STAGE) == 0)
        def _stage():
            rows = pl.ds(pl.multiple_of(base + bl, _STAGE), _STAGE)
            pltpu.sync_copy(ts_hbm.at[rows], ts8)
            pltpu.sync_copy(nv_hbm.at[rows], nv8)
            pltpu.sync_copy(mk_hbm.at[rows], mk8)

        @pl.when(bl >= 2)
        def _wait_prev():
            pltpu.make_async_copy(plane.at[slot], out_hbm.at[0],
                                  sems.at[slot]).wait()

        r = lax.rem(bl, _STAGE)

        def d_body(d, _):
            idx = jnp.full((_LANES,), d, jnp.int32)
            wd_s = plsc.load_gather(wdv, [idx])
            wv_s = plsc.load_gather(wvv, [idx])
            bias_s = plsc.load_gather(biasv, [idx])
            for off in offs:
                sl = pl.ds(off, _LANES)
                val = (ts8[r, sl] * wd_s + nv8[r, sl] * wv_s
                       + bias_s) * mk8[r, sl]
                plane[slot, d, sl] = val
            return 0

        lax.fori_loop(0, D, d_body, 0, unroll=2)

        pltpu.make_async_copy(plane.at[slot], out_hbm.at[base + bl],
                              sems.at[slot]).start()
        return 0

    lax.fori_loop(0, per_w, row_body, 0)
    for s in range(2):
        pltpu.make_async_copy(plane.at[s], out_hbm.at[0], sems.at[s]).wait()


def kernel(timestamp, numerical_value, mask, code, W_date, b_date, table,
           W_val, b_val):
    B, L = timestamp.shape
    D = W_date.shape[0]
    del code

    cp = pltpu.CompilerParams()
    if "needs_layout_passes" in pltpu.CompilerParams.__dataclass_fields__:
        cp = dataclasses.replace(cp, needs_layout_passes=False)

    run = functools.partial(
        pl.kernel,
        out_type=jax.ShapeDtypeStruct((B, D, L), jnp.float32),
        compiler_params=cp,
        mesh=plsc.VectorSubcoreMesh(core_axis_name="c", subcore_axis_name="s"),
        scratch_types=[
            pltpu.VMEM((_STAGE, L), jnp.float32),
            pltpu.VMEM((_STAGE, L), jnp.float32),
            pltpu.VMEM((_STAGE, L), jnp.float32),
            pltpu.VMEM((D,), jnp.float32),
            pltpu.VMEM((D,), jnp.float32),
            pltpu.VMEM((D,), jnp.float32),
            pltpu.VMEM((D,), jnp.float32),
            pltpu.VMEM((D,), jnp.float32),
            pltpu.VMEM((D,), jnp.float32),
            pltpu.VMEM((2, D, L), jnp.float32),
            pltpu.SemaphoreType.DMA((2,)),
        ],
    )(_sc_body)

    return run(timestamp, numerical_value, mask.reshape(B, L),
               W_date[:, 0], W_val[:, 0], b_date, b_val, table[0, :])


# speedup vs baseline: 1.8604x; 1.8604x over previous
import dataclasses
import functools

import jax
import jax.numpy as jnp
from jax import lax
from jax.experimental import pallas as pl
from jax.experimental.pallas import tpu as pltpu
from jax.experimental.pallas import tpu_sc as plsc

_NC = 2
_NS = 16
_LANES = 16
_STAGE = 8


def _sc_body(ts_hbm, nv_hbm, mk_hbm, wd_hbm, wv_hbm, bd_hbm, bv_hbm, tb_hbm,
             out_hbm, ts8, nv8, mk8, wdv, wvv, b1, b2, b3, biasv, plane,
             sems):
    B = ts_hbm.shape[0]
    D = wd_hbm.shape[0]
    L = ts_hbm.shape[1]
    nw = _NC * _NS
    per_w = B // nw
    wid = lax.axis_index("s") * _NC + lax.axis_index("c")
    base = wid * per_w

    pltpu.sync_copy(wd_hbm, wdv)
    pltpu.sync_copy(wv_hbm, wvv)
    pltpu.sync_copy(bd_hbm, b1)
    pltpu.sync_copy(bv_hbm, b2)
    pltpu.sync_copy(tb_hbm, b3)
    for c in range(D // _LANES):
        sl = pl.ds(c * _LANES, _LANES)
        biasv[sl] = b1[sl] + b2[sl] + b3[sl]

    offs = list(range(0, L - _LANES, _LANES)) + [L - _LANES]

    def row_body(bl, _):
        slot = lax.rem(bl, 2)

        @pl.when(lax.rem(bl, _STAGE) == 0)
        def _stage():
            rows = pl.ds(pl.multiple_of(base + bl, _STAGE), _STAGE)
            pltpu.sync_copy(ts_hbm.at[rows], ts8)
            pltpu.sync_copy(nv_hbm.at[rows], nv8)
            pltpu.sync_copy(mk_hbm.at[rows], mk8)

        @pl.when(bl >= 2)
        def _wait_prev():
            pltpu.make_async_copy(plane.at[slot], out_hbm.at[0],
                                  sems.at[slot]).wait()

        r = lax.rem(bl, _STAGE)

        tsc = [ts8[r, pl.ds(o, _LANES)] for o in offs]
        nvc = [nv8[r, pl.ds(o, _LANES)] for o in offs]
        mkc = [mk8[r, pl.ds(o, _LANES)] for o in offs]
        for d in range(D):
            idx = jnp.full((_LANES,), d, jnp.int32)
            wd_s = plsc.load_gather(wdv, [idx])
            wv_s = plsc.load_gather(wvv, [idx])
            bias_s = plsc.load_gather(biasv, [idx])
            for k, o in enumerate(offs):
                plane[slot, d, pl.ds(o, _LANES)] = (
                    tsc[k] * wd_s + nvc[k] * wv_s + bias_s) * mkc[k]

        pltpu.make_async_copy(plane.at[slot], out_hbm.at[base + bl],
                              sems.at[slot]).start()
        return 0

    lax.fori_loop(0, per_w, row_body, 0)
    for s in range(2):
        pltpu.make_async_copy(plane.at[s], out_hbm.at[0], sems.at[s]).wait()


def kernel(timestamp, numerical_value, mask, code, W_date, b_date, table,
           W_val, b_val):
    B, L = timestamp.shape
    D = W_date.shape[0]
    del code

    cp = pltpu.CompilerParams()
    if "needs_layout_passes" in pltpu.CompilerParams.__dataclass_fields__:
        cp = dataclasses.replace(cp, needs_layout_passes=False)

    run = functools.partial(
        pl.kernel,
        out_type=jax.ShapeDtypeStruct((B, D, L), jnp.float32),
        compiler_params=cp,
        mesh=plsc.VectorSubcoreMesh(core_axis_name="c", subcore_axis_name="s"),
        scratch_types=[
            pltpu.VMEM((_STAGE, L), jnp.float32),
            pltpu.VMEM((_STAGE, L), jnp.float32),
            pltpu.VMEM((_STAGE, L), jnp.float32),
            pltpu.VMEM((D,), jnp.float32),
            pltpu.VMEM((D,), jnp.float32),
            pltpu.VMEM((D,), jnp.float32),
            pltpu.VMEM((D,), jnp.float32),
            pltpu.VMEM((D,), jnp.float32),
            pltpu.VMEM((D,), jnp.float32),
            pltpu.VMEM((2, D, L), jnp.float32),
            pltpu.SemaphoreType.DMA((2,)),
        ],
    )(_sc_body)

    return run(timestamp, numerical_value, mask.reshape(B, L),
               W_date[:, 0], W_val[:, 0], b_date, b_val, table[0, :])


# speedup vs baseline: 3.6692x; 1.9723x over previous
import jax
import jax.numpy as jnp
from jax.experimental import pallas as pl

_BR = 32


def _embed_body(ts_ref, nv_ref, mk_ref, wd_ref, wv_ref, bd_ref, bv_ref,
                tb_ref, out_ref):
    D = wd_ref.shape[0]
    L = ts_ref.shape[1]
    bias = bd_ref[...] + bv_ref[...] + tb_ref[...]
    wd = wd_ref[...]
    wv = wv_ref[...]
    for c in range(_BR):
        row = slice(c, c + 1)
        ts = jnp.broadcast_to(ts_ref[row, :], (D, L))
        nv = jnp.broadcast_to(nv_ref[row, :], (D, L))
        mk = jnp.broadcast_to(mk_ref[row, :], (D, L))
        out_ref[pl.ds(c * D, D)] = (ts * wd + nv * wv + bias) * mk


def kernel(timestamp, numerical_value, mask, code, W_date, b_date, table,
           W_val, b_val):
    B, L = timestamp.shape
    D = W_date.shape[0]
    del code

    row_spec = pl.BlockSpec((_BR, L), lambda i: (i, 0))
    col_spec = pl.BlockSpec((D, 1), lambda i: (0, 0))

    out2 = pl.pallas_call(
        _embed_body,
        grid=(B // _BR,),
        in_specs=[row_spec, row_spec, row_spec,
                  col_spec, col_spec, col_spec, col_spec, col_spec],
        out_specs=pl.BlockSpec((_BR * D, L), lambda i: (i, 0)),
        out_shape=jax.ShapeDtypeStruct((B * D, L), jnp.float32),
    )(timestamp, numerical_value, mask.reshape(B, L),
      W_date, W_val,
      b_date.reshape(D, 1), b_val.reshape(D, 1), table.reshape(D, 1))
    return out2.reshape(B, D, L)


# speedup vs baseline: 4.2453x; 1.1570x over previous
import jax
import jax.numpy as jnp
from jax.experimental import pallas as pl

_BR = 128


def _embed_body(ts_ref, nv_ref, mk_ref, wd_ref, wv_ref, bd_ref, bv_ref,
                tb_ref, out_ref):
    D = wd_ref.shape[0]
    L = ts_ref.shape[1]
    bias = bd_ref[...] + bv_ref[...] + tb_ref[...]
    wd = wd_ref[...]
    wv = wv_ref[...]
    for c in range(_BR):
        row = slice(c, c + 1)
        ts = jnp.broadcast_to(ts_ref[row, :], (D, L))
        nv = jnp.broadcast_to(nv_ref[row, :], (D, L))
        mk = jnp.broadcast_to(mk_ref[row, :], (D, L))
        out_ref[pl.ds(c * D, D)] = (ts * wd + nv * wv + bias) * mk


def kernel(timestamp, numerical_value, mask, code, W_date, b_date, table,
           W_val, b_val):
    B, L = timestamp.shape
    D = W_date.shape[0]
    del code

    row_spec = pl.BlockSpec((_BR, L), lambda i: (i, 0))
    col_spec = pl.BlockSpec((D, 1), lambda i: (0, 0))

    out2 = pl.pallas_call(
        _embed_body,
        grid=(B // _BR,),
        in_specs=[row_spec, row_spec, row_spec,
                  col_spec, col_spec, col_spec, col_spec, col_spec],
        out_specs=pl.BlockSpec((_BR * D, L), lambda i: (i, 0)),
        out_shape=jax.ShapeDtypeStruct((B * D, L), jnp.float32),
    )(timestamp, numerical_value, mask.reshape(B, L),
      W_date, W_val,
      b_date.reshape(D, 1), b_val.reshape(D, 1), table.reshape(D, 1))
    return out2.reshape(B, D, L)


# speedup vs baseline: 4.2490x; 1.0009x over previous
import jax
import jax.numpy as jnp
from jax.experimental import pallas as pl

_BR = 256


def _embed_body(ts_ref, nv_ref, mk_ref, wd_ref, wv_ref, bd_ref, bv_ref,
                tb_ref, out_ref):
    D = wd_ref.shape[0]
    L = ts_ref.shape[1]
    bias = bd_ref[...] + bv_ref[...] + tb_ref[...]
    wd = wd_ref[...]
    wv = wv_ref[...]
    for c in range(_BR):
        row = slice(c, c + 1)
        ts = jnp.broadcast_to(ts_ref[row, :], (D, L))
        nv = jnp.broadcast_to(nv_ref[row, :], (D, L))
        mk = jnp.broadcast_to(mk_ref[row, :], (D, L))
        out_ref[pl.ds(c * D, D)] = (ts * wd + nv * wv + bias) * mk


def kernel(timestamp, numerical_value, mask, code, W_date, b_date, table,
           W_val, b_val):
    B, L = timestamp.shape
    D = W_date.shape[0]
    del code

    row_spec = pl.BlockSpec((_BR, L), lambda i: (i, 0))
    col_spec = pl.BlockSpec((D, 1), lambda i: (0, 0))

    out2 = pl.pallas_call(
        _embed_body,
        grid=(B // _BR,),
        in_specs=[row_spec, row_spec, row_spec,
                  col_spec, col_spec, col_spec, col_spec, col_spec],
        out_specs=pl.BlockSpec((_BR * D, L), lambda i: (i, 0)),
        out_shape=jax.ShapeDtypeStruct((B * D, L), jnp.float32),
    )(timestamp, numerical_value, mask.reshape(B, L),
      W_date, W_val,
      b_date.reshape(D, 1), b_val.reshape(D, 1), table.reshape(D, 1))
    return out2.reshape(B, D, L)
